# baseline (speedup 1.0000x reference)
import jax
import jax.numpy as jnp
from jax.experimental import pallas as pl


def _copy_block(w_ref, o_ref):
    o_ref[...] = w_ref[...]


def kernel(graph, feat, edge_feat, emb_weight):
    n, d = emb_weight.shape
    block = 10000
    return pl.pallas_call(
        _copy_block,
        grid=(n // block,),
        in_specs=[pl.BlockSpec((block, d), lambda i: (i, 0))],
        out_specs=pl.BlockSpec((block, d), lambda i: (i, 0)),
        out_shape=jax.ShapeDtypeStruct((n, d), emb_weight.dtype),
    )(emb_weight)
